# baseline (speedup 1.0000x reference)
import functools

import jax
import jax.numpy as jnp
from jax import lax
from jax.experimental import pallas as pl
from jax.experimental.pallas import tpu as pltpu
from jax.experimental.pallas import tpu_sc as plsc

TM = 128
NE = 8
K = 2
NW = 32
D = 1024
DFF = 2048
FFB = 1024


def _gating_body(x_ref, gw1_ref, gb1_ref, gw2_ref,
                 slot1_ref, slot2_ref, wt1_ref, wt2_ref, te_ref, bal_ref):
    T = x_ref.shape[0]
    NT = (T * K) // TM + NE
    x = x_ref[...]
    h = jnp.maximum(jnp.dot(x, gw1_ref[...], preferred_element_type=jnp.float32)
                    + gb1_ref[...], 0.0)
    logits = jnp.dot(h, gw2_ref[...], preferred_element_type=jnp.float32)
    m = jnp.max(logits, axis=-1, keepdims=True)
    ex = jnp.exp(logits - m)
    scores = ex / jnp.sum(ex, axis=-1, keepdims=True)

    gm = jnp.sum(scores, axis=0, keepdims=True) * (1.0 / T)
    bal_ref[0, 0] = NE * jnp.sum(gm * jnp.log(gm + 1e-8))

    lane = lax.broadcasted_iota(jnp.int32, (T, NE), 1).astype(jnp.float32)
    m1 = jnp.max(scores, axis=-1, keepdims=True)
    i1 = jnp.min(jnp.where(scores == m1, lane, float(NE)), axis=-1, keepdims=True)
    sm = jnp.where(lane == i1, -jnp.inf, scores)
    m2 = jnp.max(sm, axis=-1, keepdims=True)
    i2 = jnp.min(jnp.where(sm == m2, lane, float(NE)), axis=-1, keepdims=True)

    b = jnp.exp(m2 - m1)
    wt1 = 1.0 / (1.0 + b)
    wt2 = b * wt1
    wt1_ref[...] = jnp.broadcast_to(wt1, (T, 128))
    wt2_ref[...] = jnp.broadcast_to(wt2, (T, 128))

    oh1 = (lane == i1).astype(jnp.bfloat16)
    oh2 = (lane == i2).astype(jnp.bfloat16)
    msk = oh1 + oh2
    TB = 256
    r = lax.broadcasted_iota(jnp.int32, (TB, TB), 0)
    c = lax.broadcasted_iota(jnp.int32, (TB, TB), 1)
    tril = (r > c).astype(jnp.bfloat16)
    blocks = []
    run = jnp.zeros((1, NE), jnp.float32)
    for bi in range(T // TB):
        mb = msk[bi * TB:(bi + 1) * TB, :]
        pin = jnp.dot(tril, mb, preferred_element_type=jnp.float32)
        blocks.append(pin + run)
        run = run + pin[TB - 1:TB, :] + mb[TB - 1:TB, :].astype(jnp.float32)
    P = jnp.concatenate(blocks, axis=0)
    counts = run
    pc = jnp.ceil(counts * (1.0 / TM)) * TM
    r8 = lax.broadcasted_iota(jnp.int32, (NE, NE), 0)
    c8 = lax.broadcasted_iota(jnp.int32, (NE, NE), 1)
    tri8 = (r8 < c8).astype(jnp.float32)
    off = jnp.dot(pc, tri8, preferred_element_type=jnp.float32)
    slotf = off + P
    oh1f = oh1.astype(jnp.float32)
    oh2f = oh2.astype(jnp.float32)
    slot1_ref[...] = jnp.sum(oh1f * slotf, axis=-1, keepdims=True).astype(jnp.int32)
    slot2_ref[...] = jnp.sum(oh2f * slotf, axis=-1, keepdims=True).astype(jnp.int32)

    tstart = off * (1.0 / TM)
    ti = lax.broadcasted_iota(jnp.int32, (NT, NE), 0).astype(jnp.float32)
    te = jnp.sum((ti >= tstart).astype(jnp.int32), axis=-1, keepdims=True) - 1
    te_ref[...] = jnp.minimum(te, NE - 1)


def _gating(x_flat, gate_w1, gate_b1, gate_w2):
    T = x_flat.shape[0]
    return pl.pallas_call(
        _gating_body,
        out_shape=(
            jax.ShapeDtypeStruct((T, 1), jnp.int32),
            jax.ShapeDtypeStruct((T, 1), jnp.int32),
            jax.ShapeDtypeStruct((T, 128), jnp.float32),
            jax.ShapeDtypeStruct((T, 128), jnp.float32),
            jax.ShapeDtypeStruct(((T * K) // TM + NE, 1), jnp.int32),
            jax.ShapeDtypeStruct((1, 1), jnp.float32),
        ),
        out_specs=(
            pl.BlockSpec(memory_space=pltpu.VMEM),
            pl.BlockSpec(memory_space=pltpu.VMEM),
            pl.BlockSpec(memory_space=pltpu.VMEM),
            pl.BlockSpec(memory_space=pltpu.VMEM),
            pl.BlockSpec(memory_space=pltpu.VMEM),
            pl.BlockSpec(memory_space=pltpu.SMEM),
        ),
    )(x_flat, gate_w1, gate_b1.reshape(1, -1), gate_w2)


def _ffn_body(te_ref, xd_ref, w1_ref, w2_ref, w3_ref, ws_ref, y_ref):
    x = xd_ref[...]
    a = jnp.dot(x, w1_ref[0], preferred_element_type=jnp.float32)
    b = jnp.dot(x, w2_ref[0], preferred_element_type=jnp.float32)
    hh = (a * jax.nn.sigmoid(a)) * b
    y = jnp.dot(hh, w3_ref[0], preferred_element_type=jnp.float32)
    y_ref[...] = y * ws_ref[:, 0:1]


def _ffn(xd, w1, w2, w3, wslot, tile_eid, nt):
    total = xd.shape[0]
    return pl.pallas_call(
        _ffn_body,
        grid_spec=pltpu.PrefetchScalarGridSpec(
            num_scalar_prefetch=1,
            grid=(nt,),
            in_specs=[
                pl.BlockSpec((TM, D), lambda i, te: (i, 0)),
                pl.BlockSpec((1, D, DFF), lambda i, te: (te[i], 0, 0)),
                pl.BlockSpec((1, D, DFF), lambda i, te: (te[i], 0, 0)),
                pl.BlockSpec((1, DFF, D), lambda i, te: (te[i], 0, 0)),
                pl.BlockSpec((TM, 128), lambda i, te: (i, 0)),
            ],
            out_specs=pl.BlockSpec((TM, D), lambda i, te: (i, 0)),
        ),
        out_shape=jax.ShapeDtypeStruct((total, D), jnp.float32),
    )(tile_eid, xd, w1, w2, w3, wslot)


def _dispatch(x_flat, slot1, slot2, wt1x, wt2x, total):
    T = x_flat.shape[0]
    per_w = T // NW
    CH = 32
    mesh = plsc.VectorSubcoreMesh(core_axis_name="c", subcore_axis_name="s")

    nch = per_w // CH

    @functools.partial(
        pl.kernel, mesh=mesh,
        out_type=(
            jax.ShapeDtypeStruct((total, D), jnp.float32),
            jax.ShapeDtypeStruct((total, 128), jnp.float32),
        ),
        scratch_types=[
            [pltpu.VMEM((CH,), jnp.int32) for _ in range(nch)],
            [pltpu.VMEM((CH,), jnp.int32) for _ in range(nch)],
            [pltpu.VMEM((CH, D), jnp.float32) for _ in range(nch)],
            [pltpu.VMEM((CH, 128), jnp.float32) for _ in range(nch)],
            [pltpu.VMEM((CH, 128), jnp.float32) for _ in range(nch)],
            [pltpu.SemaphoreType.DMA for _ in range(nch)],
            [pltpu.SemaphoreType.DMA for _ in range(nch)],
            [pltpu.SemaphoreType.DMA for _ in range(nch)],
            [pltpu.SemaphoreType.DMA for _ in range(nch)],
        ],
    )
    def disp(x_hbm, s1_hbm, s2_hbm, wt1_hbm, wt2_hbm, xd_hbm, ws_hbm,
             s1_vs, s2_vs, rows_vs, w1_vs, w2_vs, sems1, sems2, semw1, semw2):
        wid = lax.axis_index("s") * 2 + lax.axis_index("c")
        pend = []
        for cidx in range(nch):
            base = wid * per_w + cidx * CH
            pltpu.sync_copy(s1_hbm.at[pl.ds(base, CH)], s1_vs[cidx])
            pltpu.sync_copy(s2_hbm.at[pl.ds(base, CH)], s2_vs[cidx])
            pltpu.sync_copy(x_hbm.at[pl.ds(base, CH)], rows_vs[cidx])
            pltpu.sync_copy(wt1_hbm.at[pl.ds(base, CH)], w1_vs[cidx])
            pltpu.sync_copy(wt2_hbm.at[pl.ds(base, CH)], w2_vs[cidx])
            pend.append(pltpu.async_copy(rows_vs[cidx], xd_hbm.at[s1_vs[cidx]], sems1[cidx]))
            pend.append(pltpu.async_copy(rows_vs[cidx], xd_hbm.at[s2_vs[cidx]], sems2[cidx]))
            pend.append(pltpu.async_copy(w1_vs[cidx], ws_hbm.at[s1_vs[cidx]], semw1[cidx]))
            pend.append(pltpu.async_copy(w2_vs[cidx], ws_hbm.at[s2_vs[cidx]], semw2[cidx]))
        for cp in pend:
            cp.wait()

    return disp(x_flat, slot1, slot2, wt1x, wt2x)


def _combine(y, slot1, slot2, T):
    per_w = T // NW
    CH = 16
    mesh = plsc.VectorSubcoreMesh(core_axis_name="c", subcore_axis_name="s")

    nch = per_w // CH

    @functools.partial(
        pl.kernel, mesh=mesh,
        out_type=jax.ShapeDtypeStruct((T, D), jnp.float32),
        scratch_types=[
            pltpu.VMEM((per_w,), jnp.int32),
            pltpu.VMEM((per_w,), jnp.int32),
            pltpu.VMEM((CH, D), jnp.float32),
            pltpu.VMEM((CH, D), jnp.float32),
            pltpu.VMEM((CH, D), jnp.float32),
            pltpu.VMEM((CH, D), jnp.float32),
            pltpu.VMEM((CH, D), jnp.float32),
            pltpu.VMEM((CH, D), jnp.float32),
            pltpu.SemaphoreType.DMA,
            pltpu.SemaphoreType.DMA,
            pltpu.SemaphoreType.DMA,
            pltpu.SemaphoreType.DMA,
            pltpu.SemaphoreType.DMA,
            pltpu.SemaphoreType.DMA,
        ],
    )
    def comb(y_hbm, s1_hbm, s2_hbm, out_hbm,
             s1_v, s2_v, b1a, b2a, b1b, b2b, oba, obb,
             g1a, g2a, g1b, g2b, sta, stb):
        wid = lax.axis_index("s") * 2 + lax.axis_index("c")
        base0 = wid * per_w
        pltpu.sync_copy(s1_hbm.at[pl.ds(base0, per_w)], s1_v)
        pltpu.sync_copy(s2_hbm.at[pl.ds(base0, per_w)], s2_v)
        bufs = [(b1a, b2a, oba, g1a, g2a, sta), (b1b, b2b, obb, g1b, g2b, stb)]

        def start(c):
            b1, b2, _, g1, g2, _ = bufs[c % 2]
            cp1 = pltpu.async_copy(y_hbm.at[s1_v.at[pl.ds(c * CH, CH)]], b1, g1)
            cp2 = pltpu.async_copy(y_hbm.at[s2_v.at[pl.ds(c * CH, CH)]], b2, g2)
            return cp1, cp2

        pend = start(0)
        stores = [None, None]
        for c in range(nch):
            nxt = start(c + 1) if c + 1 < nch else None
            pend[0].wait()
            pend[1].wait()
            b1, b2, ob, _, _, stsem = bufs[c % 2]
            if stores[c % 2] is not None:
                stores[c % 2].wait()
            for i in range(CH):

                def body(jj, _, i=i, b1=b1, b2=b2, ob=ob):
                    col = jj * 64
                    for u in range(4):
                        sl = pl.ds(col + u * 16, 16)
                        ob[i, sl] = b1[i, sl] + b2[i, sl]
                    return 0

                lax.fori_loop(0, D // 64, body, 0)
            st = pltpu.async_copy(ob, out_hbm.at[pl.ds(base0 + c * CH, CH)], stsem)
            stores[c % 2] = st
            pend = nxt
        for st in stores:
            if st is not None:
                st.wait()

    return comb(y, slot1, slot2)


def kernel(x, gate_w1, gate_b1, gate_w2, w1, w2, w3):
    b, s, d = x.shape
    T = b * s
    NT = (T * K) // TM + NE
    total = NT * TM
    x_flat = x.reshape(T, d)

    slot1, slot2, wt1x, wt2x, te, bal = _gating(x_flat, gate_w1, gate_b1, gate_w2)
    slot1 = slot1.reshape(T)
    slot2 = slot2.reshape(T)
    tile_eid = te.reshape(NT)

    xd, wslot = _dispatch(x_flat, slot1, slot2, wt1x, wt2x, total)
    y = _ffn(xd, w1, w2, w3, wslot, tile_eid, NT)
    out = _combine(y, slot1, slot2, T)
    return out.reshape(b, s, d), bal[0, 0]


# speedup vs baseline: 1.0946x; 1.0946x over previous
import functools

import jax
import jax.numpy as jnp
from jax import lax
from jax.experimental import pallas as pl
from jax.experimental.pallas import tpu as pltpu
from jax.experimental.pallas import tpu_sc as plsc

TM = 256
NE = 8
K = 2
NW = 32
D = 1024
DFF = 2048
FFB = 1024


def _gating_body(x_ref, gw1_ref, gb1_ref, gw2_ref,
                 slot1_ref, slot2_ref, wt1_ref, wt2_ref, te_ref, bal_ref):
    T = x_ref.shape[0]
    NT = (T * K) // TM + NE
    x = x_ref[...]
    h = jnp.maximum(jnp.dot(x, gw1_ref[...], preferred_element_type=jnp.float32)
                    + gb1_ref[...], 0.0)
    logits = jnp.dot(h, gw2_ref[...], preferred_element_type=jnp.float32)
    m = jnp.max(logits, axis=-1, keepdims=True)
    ex = jnp.exp(logits - m)
    scores = ex / jnp.sum(ex, axis=-1, keepdims=True)

    gm = jnp.sum(scores, axis=0, keepdims=True) * (1.0 / T)
    bal_ref[0, 0] = NE * jnp.sum(gm * jnp.log(gm + 1e-8))

    lane = lax.broadcasted_iota(jnp.int32, (T, NE), 1).astype(jnp.float32)
    m1 = jnp.max(scores, axis=-1, keepdims=True)
    i1 = jnp.min(jnp.where(scores == m1, lane, float(NE)), axis=-1, keepdims=True)
    sm = jnp.where(lane == i1, -jnp.inf, scores)
    m2 = jnp.max(sm, axis=-1, keepdims=True)
    i2 = jnp.min(jnp.where(sm == m2, lane, float(NE)), axis=-1, keepdims=True)

    b = jnp.exp(m2 - m1)
    wt1 = 1.0 / (1.0 + b)
    wt2 = b * wt1
    wt1_ref[...] = jnp.broadcast_to(wt1, (T, 128))
    wt2_ref[...] = jnp.broadcast_to(wt2, (T, 128))

    oh1 = (lane == i1).astype(jnp.bfloat16)
    oh2 = (lane == i2).astype(jnp.bfloat16)
    msk = oh1 + oh2
    TB = 256
    r = lax.broadcasted_iota(jnp.int32, (TB, TB), 0)
    c = lax.broadcasted_iota(jnp.int32, (TB, TB), 1)
    tril = (r > c).astype(jnp.bfloat16)
    blocks = []
    run = jnp.zeros((1, NE), jnp.float32)
    for bi in range(T // TB):
        mb = msk[bi * TB:(bi + 1) * TB, :]
        pin = jnp.dot(tril, mb, preferred_element_type=jnp.float32)
        blocks.append(pin + run)
        run = run + pin[TB - 1:TB, :] + mb[TB - 1:TB, :].astype(jnp.float32)
    P = jnp.concatenate(blocks, axis=0)
    counts = run
    pc = jnp.ceil(counts * (1.0 / TM)) * TM
    r8 = lax.broadcasted_iota(jnp.int32, (NE, NE), 0)
    c8 = lax.broadcasted_iota(jnp.int32, (NE, NE), 1)
    tri8 = (r8 < c8).astype(jnp.float32)
    off = jnp.dot(pc, tri8, preferred_element_type=jnp.float32)
    slotf = off + P
    oh1f = oh1.astype(jnp.float32)
    oh2f = oh2.astype(jnp.float32)
    slot1_ref[...] = jnp.sum(oh1f * slotf, axis=-1, keepdims=True).astype(jnp.int32)
    slot2_ref[...] = jnp.sum(oh2f * slotf, axis=-1, keepdims=True).astype(jnp.int32)

    tstart = off * (1.0 / TM)
    ti = lax.broadcasted_iota(jnp.int32, (NT, NE), 0).astype(jnp.float32)
    te = jnp.sum((ti >= tstart).astype(jnp.int32), axis=-1, keepdims=True) - 1
    te_ref[...] = jnp.minimum(te, NE - 1)


def _gating(x_flat, gate_w1, gate_b1, gate_w2):
    T = x_flat.shape[0]
    return pl.pallas_call(
        _gating_body,
        out_shape=(
            jax.ShapeDtypeStruct((T, 1), jnp.int32),
            jax.ShapeDtypeStruct((T, 1), jnp.int32),
            jax.ShapeDtypeStruct((T, 128), jnp.float32),
            jax.ShapeDtypeStruct((T, 128), jnp.float32),
            jax.ShapeDtypeStruct(((T * K) // TM + NE, 1), jnp.int32),
            jax.ShapeDtypeStruct((1, 1), jnp.float32),
        ),
        out_specs=(
            pl.BlockSpec(memory_space=pltpu.VMEM),
            pl.BlockSpec(memory_space=pltpu.VMEM),
            pl.BlockSpec(memory_space=pltpu.VMEM),
            pl.BlockSpec(memory_space=pltpu.VMEM),
            pl.BlockSpec(memory_space=pltpu.VMEM),
            pl.BlockSpec(memory_space=pltpu.SMEM),
        ),
    )(x_flat, gate_w1, gate_b1.reshape(1, -1), gate_w2)


def _ffn_body(meta_ref, xd_ref, ws_ref, w1_any, w2_any, w3_any, y_ref,
              w1b0, w2b0, w3b0, w1b1, w2b1, w3b1,
              s10, s20, s30, s11, s21, s31):
    i = pl.program_id(0)
    first = meta_ref[i, 1]
    parity = meta_ref[i, 2]
    nxt = meta_ref[i, 3]
    bufs = ((w1b0, w2b0, w3b0, s10, s20, s30),
            (w1b1, w2b1, w3b1, s11, s21, s31))

    def start_copies(eid, bset):
        w1b, w2b, w3b, sa, sb, sc = bset
        pltpu.make_async_copy(w1_any.at[eid], w1b, sa).start()
        pltpu.make_async_copy(w2_any.at[eid], w2b, sb).start()
        pltpu.make_async_copy(w3_any.at[eid], w3b, sc).start()

    def wait_copies(bset):
        w1b, w2b, w3b, sa, sb, sc = bset
        pltpu.make_async_copy(w1_any.at[0], w1b, sa).wait()
        pltpu.make_async_copy(w2_any.at[0], w2b, sb).wait()
        pltpu.make_async_copy(w3_any.at[0], w3b, sc).wait()

    @pl.when(first == 1)
    def _():
        @pl.when(i == 0)
        def _():
            start_copies(meta_ref[0, 0], bufs[0])

        @pl.when(parity == 0)
        def _():
            wait_copies(bufs[0])

        @pl.when(parity == 1)
        def _():
            wait_copies(bufs[1])

        @pl.when((nxt >= 0) & (parity == 0))
        def _():
            start_copies(nxt, bufs[1])

        @pl.when((nxt >= 0) & (parity == 1))
        def _():
            start_copies(nxt, bufs[0])

    def compute(bset):
        w1b, w2b, w3b = bset[0], bset[1], bset[2]
        x = xd_ref[...]
        a = jnp.dot(x, w1b[...], preferred_element_type=jnp.float32)
        b = jnp.dot(x, w2b[...], preferred_element_type=jnp.float32)
        hh = (a * jax.nn.sigmoid(a)) * b
        y = jnp.dot(hh, w3b[...], preferred_element_type=jnp.float32)
        y_ref[...] = y * ws_ref[:, 0:1]

    @pl.when(parity == 0)
    def _():
        compute(bufs[0])

    @pl.when(parity == 1)
    def _():
        compute(bufs[1])


def _ffn(xd, w1, w2, w3, wslot, meta, nt):
    total = xd.shape[0]
    return pl.pallas_call(
        _ffn_body,
        grid_spec=pltpu.PrefetchScalarGridSpec(
            num_scalar_prefetch=1,
            grid=(nt,),
            in_specs=[
                pl.BlockSpec((TM, D), lambda i, m: (i, 0)),
                pl.BlockSpec((TM, 128), lambda i, m: (i, 0)),
                pl.BlockSpec(memory_space=pl.ANY),
                pl.BlockSpec(memory_space=pl.ANY),
                pl.BlockSpec(memory_space=pl.ANY),
            ],
            out_specs=pl.BlockSpec((TM, D), lambda i, m: (i, 0)),
            scratch_shapes=[
                pltpu.VMEM((D, DFF), jnp.float32),
                pltpu.VMEM((D, DFF), jnp.float32),
                pltpu.VMEM((DFF, D), jnp.float32),
                pltpu.VMEM((D, DFF), jnp.float32),
                pltpu.VMEM((D, DFF), jnp.float32),
                pltpu.VMEM((DFF, D), jnp.float32),
                pltpu.SemaphoreType.DMA,
                pltpu.SemaphoreType.DMA,
                pltpu.SemaphoreType.DMA,
                pltpu.SemaphoreType.DMA,
                pltpu.SemaphoreType.DMA,
                pltpu.SemaphoreType.DMA,
            ],
        ),
        out_shape=jax.ShapeDtypeStruct((total, D), jnp.float32),
    )(meta, xd, wslot, w1, w2, w3)


def _dispatch(x_flat, slot1, slot2, wt1x, wt2x, total):
    T = x_flat.shape[0]
    per_w = T // NW
    CH = 32
    mesh = plsc.VectorSubcoreMesh(core_axis_name="c", subcore_axis_name="s")

    nch = per_w // CH

    @functools.partial(
        pl.kernel, mesh=mesh,
        out_type=(
            jax.ShapeDtypeStruct((total, D), jnp.float32),
            jax.ShapeDtypeStruct((total, 128), jnp.float32),
        ),
        scratch_types=[
            [pltpu.VMEM((CH,), jnp.int32) for _ in range(nch)],
            [pltpu.VMEM((CH,), jnp.int32) for _ in range(nch)],
            [pltpu.VMEM((CH, D), jnp.float32) for _ in range(nch)],
            [pltpu.VMEM((CH, 128), jnp.float32) for _ in range(nch)],
            [pltpu.VMEM((CH, 128), jnp.float32) for _ in range(nch)],
            [pltpu.SemaphoreType.DMA for _ in range(nch)],
            [pltpu.SemaphoreType.DMA for _ in range(nch)],
            [pltpu.SemaphoreType.DMA for _ in range(nch)],
            [pltpu.SemaphoreType.DMA for _ in range(nch)],
        ],
    )
    def disp(x_hbm, s1_hbm, s2_hbm, wt1_hbm, wt2_hbm, xd_hbm, ws_hbm,
             s1_vs, s2_vs, rows_vs, w1_vs, w2_vs, sems1, sems2, semw1, semw2):
        wid = lax.axis_index("s") * 2 + lax.axis_index("c")
        pend = []
        for cidx in range(nch):
            base = wid * per_w + cidx * CH
            pltpu.sync_copy(s1_hbm.at[pl.ds(base, CH)], s1_vs[cidx])
            pltpu.sync_copy(s2_hbm.at[pl.ds(base, CH)], s2_vs[cidx])
            pltpu.sync_copy(x_hbm.at[pl.ds(base, CH)], rows_vs[cidx])
            pltpu.sync_copy(wt1_hbm.at[pl.ds(base, CH)], w1_vs[cidx])
            pltpu.sync_copy(wt2_hbm.at[pl.ds(base, CH)], w2_vs[cidx])
            pend.append(pltpu.async_copy(rows_vs[cidx], xd_hbm.at[s1_vs[cidx]], sems1[cidx]))
            pend.append(pltpu.async_copy(rows_vs[cidx], xd_hbm.at[s2_vs[cidx]], sems2[cidx]))
            pend.append(pltpu.async_copy(w1_vs[cidx], ws_hbm.at[s1_vs[cidx]], semw1[cidx]))
            pend.append(pltpu.async_copy(w2_vs[cidx], ws_hbm.at[s2_vs[cidx]], semw2[cidx]))
        for cp in pend:
            cp.wait()

    return disp(x_flat, slot1, slot2, wt1x, wt2x)


def _combine(y, slot1, slot2, T):
    per_w = T // NW
    CH = 16
    mesh = plsc.VectorSubcoreMesh(core_axis_name="c", subcore_axis_name="s")

    nch = per_w // CH

    @functools.partial(
        pl.kernel, mesh=mesh,
        out_type=jax.ShapeDtypeStruct((T, D), jnp.float32),
        scratch_types=[
            pltpu.VMEM((per_w,), jnp.int32),
            pltpu.VMEM((per_w,), jnp.int32),
            pltpu.VMEM((CH, D), jnp.float32),
            pltpu.VMEM((CH, D), jnp.float32),
            pltpu.VMEM((CH, D), jnp.float32),
            pltpu.VMEM((CH, D), jnp.float32),
            pltpu.VMEM((CH, D), jnp.float32),
            pltpu.VMEM((CH, D), jnp.float32),
            pltpu.SemaphoreType.DMA,
            pltpu.SemaphoreType.DMA,
            pltpu.SemaphoreType.DMA,
            pltpu.SemaphoreType.DMA,
            pltpu.SemaphoreType.DMA,
            pltpu.SemaphoreType.DMA,
        ],
    )
    def comb(y_hbm, s1_hbm, s2_hbm, out_hbm,
             s1_v, s2_v, b1a, b2a, b1b, b2b, oba, obb,
             g1a, g2a, g1b, g2b, sta, stb):
        wid = lax.axis_index("s") * 2 + lax.axis_index("c")
        base0 = wid * per_w
        pltpu.sync_copy(s1_hbm.at[pl.ds(base0, per_w)], s1_v)
        pltpu.sync_copy(s2_hbm.at[pl.ds(base0, per_w)], s2_v)
        bufs = [(b1a, b2a, oba, g1a, g2a, sta), (b1b, b2b, obb, g1b, g2b, stb)]

        def start(c):
            b1, b2, _, g1, g2, _ = bufs[c % 2]
            cp1 = pltpu.async_copy(y_hbm.at[s1_v.at[pl.ds(c * CH, CH)]], b1, g1)
            cp2 = pltpu.async_copy(y_hbm.at[s2_v.at[pl.ds(c * CH, CH)]], b2, g2)
            return cp1, cp2

        pend = start(0)
        stores = [None, None]
        for c in range(nch):
            nxt = start(c + 1) if c + 1 < nch else None
            pend[0].wait()
            pend[1].wait()
            b1, b2, ob, _, _, stsem = bufs[c % 2]
            if stores[c % 2] is not None:
                stores[c % 2].wait()
            for i in range(CH):

                def body(jj, _, i=i, b1=b1, b2=b2, ob=ob):
                    col = jj * 64
                    for u in range(4):
                        sl = pl.ds(col + u * 16, 16)
                        ob[i, sl] = b1[i, sl] + b2[i, sl]
                    return 0

                lax.fori_loop(0, D // 64, body, 0)
            st = pltpu.async_copy(ob, out_hbm.at[pl.ds(base0 + c * CH, CH)], stsem)
            stores[c % 2] = st
            pend = nxt
        for st in stores:
            if st is not None:
                st.wait()

    return comb(y, slot1, slot2)


def kernel(x, gate_w1, gate_b1, gate_w2, w1, w2, w3):
    b, s, d = x.shape
    T = b * s
    NT = (T * K) // TM + NE
    total = NT * TM
    x_flat = x.reshape(T, d)

    slot1, slot2, wt1x, wt2x, te, bal = _gating(x_flat, gate_w1, gate_b1, gate_w2)
    slot1 = slot1.reshape(T)
    slot2 = slot2.reshape(T)
    tile_eid = te.reshape(NT)

    first = jnp.concatenate([jnp.ones((1,), jnp.int32),
                             (tile_eid[1:] != tile_eid[:-1]).astype(jnp.int32)])
    seg = jnp.cumsum(first) - 1
    parity = seg % 2
    ge = jnp.zeros((NT + 1,), jnp.int32).at[seg].max(tile_eid)
    nxt = jnp.where(seg + 1 <= seg[NT - 1], ge[seg + 1], -1)
    meta = jnp.stack([tile_eid, first, parity, nxt], axis=1)

    xd, wslot = _dispatch(x_flat, slot1, slot2, wt1x, wt2x, total)
    y = _ffn(xd, w1, w2, w3, wslot, meta, NT)
    out = _combine(y, slot1, slot2, T)
    return out.reshape(b, s, d), bal[0, 0]


# speedup vs baseline: 1.1780x; 1.0762x over previous
import functools

import jax
import jax.numpy as jnp
from jax import lax
from jax.experimental import pallas as pl
from jax.experimental.pallas import tpu as pltpu
from jax.experimental.pallas import tpu_sc as plsc

TM = 256
NE = 8
K = 2
NW = 32
D = 1024
DFF = 2048
FFB = 1024


def _gating_body(x_ref, gw1_ref, gb1_ref, gw2_ref,
                 slot1_ref, slot2_ref, wt1_ref, wt2_ref, te_ref, bal_ref):
    T = x_ref.shape[0]
    NT = (T * K) // TM + NE
    x = x_ref[...]
    h = jnp.maximum(jnp.dot(x, gw1_ref[...], preferred_element_type=jnp.float32)
                    + gb1_ref[...], 0.0)
    logits = jnp.dot(h, gw2_ref[...], preferred_element_type=jnp.float32)
    m = jnp.max(logits, axis=-1, keepdims=True)
    ex = jnp.exp(logits - m)
    scores = ex / jnp.sum(ex, axis=-1, keepdims=True)

    gm = jnp.sum(scores, axis=0, keepdims=True) * (1.0 / T)
    bal_ref[0, 0] = NE * jnp.sum(gm * jnp.log(gm + 1e-8))

    lane = lax.broadcasted_iota(jnp.int32, (T, NE), 1).astype(jnp.float32)
    m1 = jnp.max(scores, axis=-1, keepdims=True)
    i1 = jnp.min(jnp.where(scores == m1, lane, float(NE)), axis=-1, keepdims=True)
    sm = jnp.where(lane == i1, -jnp.inf, scores)
    m2 = jnp.max(sm, axis=-1, keepdims=True)
    i2 = jnp.min(jnp.where(sm == m2, lane, float(NE)), axis=-1, keepdims=True)

    b = jnp.exp(m2 - m1)
    wt1 = 1.0 / (1.0 + b)
    wt2 = b * wt1
    wt1_ref[...] = jnp.broadcast_to(wt1, (T, 128))
    wt2_ref[...] = jnp.broadcast_to(wt2, (T, 128))

    oh1 = (lane == i1).astype(jnp.bfloat16)
    oh2 = (lane == i2).astype(jnp.bfloat16)
    msk = oh1 + oh2
    TB = 256
    r = lax.broadcasted_iota(jnp.int32, (TB, TB), 0)
    c = lax.broadcasted_iota(jnp.int32, (TB, TB), 1)
    tril = (r > c).astype(jnp.bfloat16)
    blocks = []
    run = jnp.zeros((1, NE), jnp.float32)
    for bi in range(T // TB):
        mb = msk[bi * TB:(bi + 1) * TB, :]
        pin = jnp.dot(tril, mb, preferred_element_type=jnp.float32)
        blocks.append(pin + run)
        run = run + pin[TB - 1:TB, :] + mb[TB - 1:TB, :].astype(jnp.float32)
    P = jnp.concatenate(blocks, axis=0)
    counts = run
    pc = jnp.ceil(counts * (1.0 / TM)) * TM
    r8 = lax.broadcasted_iota(jnp.int32, (NE, NE), 0)
    c8 = lax.broadcasted_iota(jnp.int32, (NE, NE), 1)
    tri8 = (r8 < c8).astype(jnp.float32)
    off = jnp.dot(pc, tri8, preferred_element_type=jnp.float32)
    slotf = off + P
    oh1f = oh1.astype(jnp.float32)
    oh2f = oh2.astype(jnp.float32)
    slot1_ref[...] = jnp.sum(oh1f * slotf, axis=-1, keepdims=True).astype(jnp.int32)
    slot2_ref[...] = jnp.sum(oh2f * slotf, axis=-1, keepdims=True).astype(jnp.int32)

    tstart = off * (1.0 / TM)
    ti = lax.broadcasted_iota(jnp.int32, (NT, NE), 0).astype(jnp.float32)
    te = jnp.sum((ti >= tstart).astype(jnp.int32), axis=-1, keepdims=True) - 1
    used = jnp.sum(pc) * (1.0 / TM)
    active = (ti[:, 0:1] < used).astype(jnp.int32)
    te_ref[...] = jnp.concatenate([jnp.minimum(te, NE - 1), active], axis=1)


def _gating(x_flat, gate_w1, gate_b1, gate_w2):
    T = x_flat.shape[0]
    return pl.pallas_call(
        _gating_body,
        out_shape=(
            jax.ShapeDtypeStruct((T, 1), jnp.int32),
            jax.ShapeDtypeStruct((T, 1), jnp.int32),
            jax.ShapeDtypeStruct((T, 128), jnp.float32),
            jax.ShapeDtypeStruct((T, 128), jnp.float32),
            jax.ShapeDtypeStruct(((T * K) // TM + NE, 2), jnp.int32),
            jax.ShapeDtypeStruct((1, 1), jnp.float32),
        ),
        out_specs=(
            pl.BlockSpec(memory_space=pltpu.VMEM),
            pl.BlockSpec(memory_space=pltpu.VMEM),
            pl.BlockSpec(memory_space=pltpu.VMEM),
            pl.BlockSpec(memory_space=pltpu.VMEM),
            pl.BlockSpec(memory_space=pltpu.VMEM),
            pl.BlockSpec(memory_space=pltpu.SMEM),
        ),
    )(x_flat, gate_w1, gate_b1.reshape(1, -1), gate_w2)


def _ffn_body(meta_ref, xd_ref, ws_ref, w1_any, w2_any, w3_any, y_ref,
              w1b0, w2b0, w3b0, w1b1, w2b1, w3b1,
              s10, s20, s30, s11, s21, s31):
    i = pl.program_id(0)
    first = meta_ref[i, 1]
    parity = meta_ref[i, 2]
    nxt = meta_ref[i, 3]
    bufs = ((w1b0, w2b0, w3b0, s10, s20, s30),
            (w1b1, w2b1, w3b1, s11, s21, s31))

    def start_copies(eid, bset):
        w1b, w2b, w3b, sa, sb, sc = bset
        pltpu.make_async_copy(w1_any.at[eid], w1b, sa).start()
        pltpu.make_async_copy(w2_any.at[eid], w2b, sb).start()
        pltpu.make_async_copy(w3_any.at[eid], w3b, sc).start()

    def wait_copies(bset):
        w1b, w2b, w3b, sa, sb, sc = bset
        pltpu.make_async_copy(w1_any.at[0], w1b, sa).wait()
        pltpu.make_async_copy(w2_any.at[0], w2b, sb).wait()
        pltpu.make_async_copy(w3_any.at[0], w3b, sc).wait()

    @pl.when(first == 1)
    def _():
        @pl.when(i == 0)
        def _():
            start_copies(meta_ref[0, 0], bufs[0])

        @pl.when(parity == 0)
        def _():
            wait_copies(bufs[0])

        @pl.when(parity == 1)
        def _():
            wait_copies(bufs[1])

        @pl.when((nxt >= 0) & (parity == 0))
        def _():
            start_copies(nxt, bufs[1])

        @pl.when((nxt >= 0) & (parity == 1))
        def _():
            start_copies(nxt, bufs[0])

    def compute(bset):
        w1b, w2b, w3b = bset[0], bset[1], bset[2]
        x = xd_ref[...]
        a = jnp.dot(x, w1b[...], preferred_element_type=jnp.float32)
        b = jnp.dot(x, w2b[...], preferred_element_type=jnp.float32)
        hh = (a * jax.nn.sigmoid(a)) * b
        y = jnp.dot(hh, w3b[...], preferred_element_type=jnp.float32)
        y_ref[...] = y * ws_ref[:, 0:1]

    active = meta_ref[i, 4]

    @pl.when((parity == 0) & (active == 1))
    def _():
        compute(bufs[0])

    @pl.when((parity == 1) & (active == 1))
    def _():
        compute(bufs[1])


def _ffn(xd, w1, w2, w3, wslot, meta, nt):
    total = xd.shape[0]
    return pl.pallas_call(
        _ffn_body,
        grid_spec=pltpu.PrefetchScalarGridSpec(
            num_scalar_prefetch=1,
            grid=(nt,),
            in_specs=[
                pl.BlockSpec((TM, D), lambda i, m: (i, 0)),
                pl.BlockSpec((TM, 128), lambda i, m: (i, 0)),
                pl.BlockSpec(memory_space=pl.ANY),
                pl.BlockSpec(memory_space=pl.ANY),
                pl.BlockSpec(memory_space=pl.ANY),
            ],
            out_specs=pl.BlockSpec((TM, D), lambda i, m: (i, 0)),
            scratch_shapes=[
                pltpu.VMEM((D, DFF), jnp.float32),
                pltpu.VMEM((D, DFF), jnp.float32),
                pltpu.VMEM((DFF, D), jnp.float32),
                pltpu.VMEM((D, DFF), jnp.float32),
                pltpu.VMEM((D, DFF), jnp.float32),
                pltpu.VMEM((DFF, D), jnp.float32),
                pltpu.SemaphoreType.DMA,
                pltpu.SemaphoreType.DMA,
                pltpu.SemaphoreType.DMA,
                pltpu.SemaphoreType.DMA,
                pltpu.SemaphoreType.DMA,
                pltpu.SemaphoreType.DMA,
            ],
        ),
        out_shape=jax.ShapeDtypeStruct((total, D), jnp.float32),
    )(meta, xd, wslot, w1, w2, w3)


def _dispatch(x_flat, slot1, slot2, wt1x, wt2x, total):
    T = x_flat.shape[0]
    per_w = T // NW
    CH = 32
    mesh = plsc.VectorSubcoreMesh(core_axis_name="c", subcore_axis_name="s")

    nch = per_w // CH

    @functools.partial(
        pl.kernel, mesh=mesh,
        out_type=(
            jax.ShapeDtypeStruct((total, D), jnp.float32),
            jax.ShapeDtypeStruct((total, 128), jnp.float32),
        ),
        scratch_types=[
            [pltpu.VMEM((CH,), jnp.int32) for _ in range(nch)],
            [pltpu.VMEM((CH,), jnp.int32) for _ in range(nch)],
            [pltpu.VMEM((CH, D), jnp.float32) for _ in range(nch)],
            [pltpu.VMEM((CH, 128), jnp.float32) for _ in range(nch)],
            [pltpu.VMEM((CH, 128), jnp.float32) for _ in range(nch)],
            [pltpu.SemaphoreType.DMA for _ in range(nch)],
            [pltpu.SemaphoreType.DMA for _ in range(nch)],
            [pltpu.SemaphoreType.DMA for _ in range(nch)],
            [pltpu.SemaphoreType.DMA for _ in range(nch)],
        ],
    )
    def disp(x_hbm, s1_hbm, s2_hbm, wt1_hbm, wt2_hbm, xd_hbm, ws_hbm,
             s1_vs, s2_vs, rows_vs, w1_vs, w2_vs, sems1, sems2, semw1, semw2):
        wid = lax.axis_index("s") * 2 + lax.axis_index("c")
        pend = []
        for cidx in range(nch):
            base = wid * per_w + cidx * CH
            pltpu.sync_copy(s1_hbm.at[pl.ds(base, CH)], s1_vs[cidx])
            pltpu.sync_copy(s2_hbm.at[pl.ds(base, CH)], s2_vs[cidx])
            pltpu.sync_copy(x_hbm.at[pl.ds(base, CH)], rows_vs[cidx])
            pltpu.sync_copy(wt1_hbm.at[pl.ds(base, CH)], w1_vs[cidx])
            pltpu.sync_copy(wt2_hbm.at[pl.ds(base, CH)], w2_vs[cidx])
            pend.append(pltpu.async_copy(rows_vs[cidx], xd_hbm.at[s1_vs[cidx]], sems1[cidx]))
            pend.append(pltpu.async_copy(rows_vs[cidx], xd_hbm.at[s2_vs[cidx]], sems2[cidx]))
            pend.append(pltpu.async_copy(w1_vs[cidx], ws_hbm.at[s1_vs[cidx]], semw1[cidx]))
            pend.append(pltpu.async_copy(w2_vs[cidx], ws_hbm.at[s2_vs[cidx]], semw2[cidx]))
        for cp in pend:
            cp.wait()

    return disp(x_flat, slot1, slot2, wt1x, wt2x)


def _combine(y, slot1, slot2, T):
    per_w = T // NW
    CH = 16
    mesh = plsc.VectorSubcoreMesh(core_axis_name="c", subcore_axis_name="s")

    nch = per_w // CH

    @functools.partial(
        pl.kernel, mesh=mesh,
        out_type=jax.ShapeDtypeStruct((T, D), jnp.float32),
        scratch_types=[
            pltpu.VMEM((per_w,), jnp.int32),
            pltpu.VMEM((per_w,), jnp.int32),
            pltpu.VMEM((CH, D), jnp.float32),
            pltpu.VMEM((CH, D), jnp.float32),
            pltpu.VMEM((CH, D), jnp.float32),
            pltpu.VMEM((CH, D), jnp.float32),
            pltpu.VMEM((CH, D), jnp.float32),
            pltpu.VMEM((CH, D), jnp.float32),
            pltpu.SemaphoreType.DMA,
            pltpu.SemaphoreType.DMA,
            pltpu.SemaphoreType.DMA,
            pltpu.SemaphoreType.DMA,
            pltpu.SemaphoreType.DMA,
            pltpu.SemaphoreType.DMA,
        ],
    )
    def comb(y_hbm, s1_hbm, s2_hbm, out_hbm,
             s1_v, s2_v, b1a, b2a, b1b, b2b, oba, obb,
             g1a, g2a, g1b, g2b, sta, stb):
        wid = lax.axis_index("s") * 2 + lax.axis_index("c")
        base0 = wid * per_w
        pltpu.sync_copy(s1_hbm.at[pl.ds(base0, per_w)], s1_v)
        pltpu.sync_copy(s2_hbm.at[pl.ds(base0, per_w)], s2_v)
        bufs = [(b1a, b2a, oba, g1a, g2a, sta), (b1b, b2b, obb, g1b, g2b, stb)]

        def start(c):
            b1, b2, _, g1, g2, _ = bufs[c % 2]
            cp1 = pltpu.async_copy(y_hbm.at[s1_v.at[pl.ds(c * CH, CH)]], b1, g1)
            cp2 = pltpu.async_copy(y_hbm.at[s2_v.at[pl.ds(c * CH, CH)]], b2, g2)
            return cp1, cp2

        pend = start(0)
        stores = [None, None]
        for c in range(nch):
            nxt = start(c + 1) if c + 1 < nch else None
            pend[0].wait()
            pend[1].wait()
            b1, b2, ob, _, _, stsem = bufs[c % 2]
            if stores[c % 2] is not None:
                stores[c % 2].wait()
            for i in range(CH):

                def body(jj, _, i=i, b1=b1, b2=b2, ob=ob):
                    col = jj * 64
                    for u in range(4):
                        sl = pl.ds(col + u * 16, 16)
                        ob[i, sl] = b1[i, sl] + b2[i, sl]
                    return 0

                lax.fori_loop(0, D // 64, body, 0)
            st = pltpu.async_copy(ob, out_hbm.at[pl.ds(base0 + c * CH, CH)], stsem)
            stores[c % 2] = st
            pend = nxt
        for st in stores:
            if st is not None:
                st.wait()

    return comb(y, slot1, slot2)


def kernel(x, gate_w1, gate_b1, gate_w2, w1, w2, w3):
    b, s, d = x.shape
    T = b * s
    NT = (T * K) // TM + NE
    total = NT * TM
    x_flat = x.reshape(T, d)

    slot1, slot2, wt1x, wt2x, te, bal = _gating(x_flat, gate_w1, gate_b1, gate_w2)
    slot1 = slot1.reshape(T)
    slot2 = slot2.reshape(T)
    tile_eid = te[:, 0]
    tile_act = te[:, 1]

    first = jnp.concatenate([jnp.ones((1,), jnp.int32),
                             (tile_eid[1:] != tile_eid[:-1]).astype(jnp.int32)])
    seg = jnp.cumsum(first) - 1
    parity = seg % 2
    ge = jnp.zeros((NT + 1,), jnp.int32).at[seg].max(tile_eid)
    nxt = jnp.where(seg + 1 <= seg[NT - 1], ge[seg + 1], -1)
    meta = jnp.stack([tile_eid, first, parity, nxt, tile_act], axis=1)

    xd, wslot = _dispatch(x_flat, slot1, slot2, wt1x, wt2x, total)
    y = _ffn(xd, w1, w2, w3, wslot, meta, NT)
    out = _combine(y, slot1, slot2, T)
    return out.reshape(b, s, d), bal[0, 0]


# speedup vs baseline: 1.1906x; 1.0107x over previous
import functools

import jax
import jax.numpy as jnp
from jax import lax
from jax.experimental import pallas as pl
from jax.experimental.pallas import tpu as pltpu
from jax.experimental.pallas import tpu_sc as plsc

TM = 256
NE = 8
K = 2
NW = 32
D = 1024
DFF = 2048
FFB = 1024


def _gating_body(x_ref, gw1_ref, gb1_ref, gw2_ref,
                 slot1_ref, slot2_ref, wt1_ref, wt2_ref, te_ref, bal_ref):
    T = x_ref.shape[0]
    NT = (T * K) // TM + NE
    x = x_ref[...]
    h = jnp.maximum(jnp.dot(x, gw1_ref[...], preferred_element_type=jnp.float32)
                    + gb1_ref[...], 0.0)
    logits = jnp.dot(h, gw2_ref[...], preferred_element_type=jnp.float32)
    m = jnp.max(logits, axis=-1, keepdims=True)
    ex = jnp.exp(logits - m)
    scores = ex / jnp.sum(ex, axis=-1, keepdims=True)

    gm = jnp.sum(scores, axis=0, keepdims=True) * (1.0 / T)
    bal_ref[0, 0] = NE * jnp.sum(gm * jnp.log(gm + 1e-8))

    lane = lax.broadcasted_iota(jnp.int32, (T, NE), 1).astype(jnp.float32)
    m1 = jnp.max(scores, axis=-1, keepdims=True)
    i1 = jnp.min(jnp.where(scores == m1, lane, float(NE)), axis=-1, keepdims=True)
    sm = jnp.where(lane == i1, -jnp.inf, scores)
    m2 = jnp.max(sm, axis=-1, keepdims=True)
    i2 = jnp.min(jnp.where(sm == m2, lane, float(NE)), axis=-1, keepdims=True)

    b = jnp.exp(m2 - m1)
    wt1 = 1.0 / (1.0 + b)
    wt2 = b * wt1
    wt1_ref[...] = jnp.broadcast_to(wt1, (T, 128))
    wt2_ref[...] = jnp.broadcast_to(wt2, (T, 128))

    oh1 = (lane == i1).astype(jnp.bfloat16)
    oh2 = (lane == i2).astype(jnp.bfloat16)
    msk = oh1 + oh2
    TB = 256
    r = lax.broadcasted_iota(jnp.int32, (TB, TB), 0)
    c = lax.broadcasted_iota(jnp.int32, (TB, TB), 1)
    tril = (r > c).astype(jnp.bfloat16)
    blocks = []
    run = jnp.zeros((1, NE), jnp.float32)
    for bi in range(T // TB):
        mb = msk[bi * TB:(bi + 1) * TB, :]
        pin = jnp.dot(tril, mb, preferred_element_type=jnp.float32)
        blocks.append(pin + run)
        run = run + pin[TB - 1:TB, :] + mb[TB - 1:TB, :].astype(jnp.float32)
    P = jnp.concatenate(blocks, axis=0)
    counts = run
    pc = jnp.ceil(counts * (1.0 / TM)) * TM
    r8 = lax.broadcasted_iota(jnp.int32, (NE, NE), 0)
    c8 = lax.broadcasted_iota(jnp.int32, (NE, NE), 1)
    tri8 = (r8 < c8).astype(jnp.float32)
    off = jnp.dot(pc, tri8, preferred_element_type=jnp.float32)
    slotf = off + P
    oh1f = oh1.astype(jnp.float32)
    oh2f = oh2.astype(jnp.float32)
    slot1_ref[...] = jnp.sum(oh1f * slotf, axis=-1, keepdims=True).astype(jnp.int32)
    slot2_ref[...] = jnp.sum(oh2f * slotf, axis=-1, keepdims=True).astype(jnp.int32)

    tstart = off * (1.0 / TM)
    ti = lax.broadcasted_iota(jnp.int32, (NT, NE), 0).astype(jnp.float32)
    te = jnp.sum((ti >= tstart).astype(jnp.int32), axis=-1, keepdims=True) - 1
    used = jnp.sum(pc) * (1.0 / TM)
    active = (ti[:, 0:1] < used).astype(jnp.int32)
    te_ref[...] = jnp.concatenate([jnp.minimum(te, NE - 1), active], axis=1)


def _gating(x_flat, gate_w1, gate_b1, gate_w2):
    T = x_flat.shape[0]
    return pl.pallas_call(
        _gating_body,
        out_shape=(
            jax.ShapeDtypeStruct((T, 1), jnp.int32),
            jax.ShapeDtypeStruct((T, 1), jnp.int32),
            jax.ShapeDtypeStruct((T, 128), jnp.float32),
            jax.ShapeDtypeStruct((T, 128), jnp.float32),
            jax.ShapeDtypeStruct(((T * K) // TM + NE, 2), jnp.int32),
            jax.ShapeDtypeStruct((1, 1), jnp.float32),
        ),
        out_specs=(
            pl.BlockSpec(memory_space=pltpu.VMEM),
            pl.BlockSpec(memory_space=pltpu.VMEM),
            pl.BlockSpec(memory_space=pltpu.VMEM),
            pl.BlockSpec(memory_space=pltpu.VMEM),
            pl.BlockSpec(memory_space=pltpu.VMEM),
            pl.BlockSpec(memory_space=pltpu.SMEM),
        ),
    )(x_flat, gate_w1, gate_b1.reshape(1, -1), gate_w2)


def _ffn_body(meta_ref, xd_ref, ws_ref, w1_any, w2_any, w3_any, y_ref,
              w1b0, w2b0, w3b0, w1b1, w2b1, w3b1,
              s10, s20, s30, s11, s21, s31):
    i = pl.program_id(0)
    first = meta_ref[i, 1]
    parity = meta_ref[i, 2]
    nxt = meta_ref[i, 3]
    bufs = ((w1b0, w2b0, w3b0, s10, s20, s30),
            (w1b1, w2b1, w3b1, s11, s21, s31))

    def start_copies(eid, bset):
        w1b, w2b, w3b, sa, sb, sc = bset
        pltpu.make_async_copy(w1_any.at[eid], w1b, sa).start()
        pltpu.make_async_copy(w2_any.at[eid], w2b, sb).start()
        pltpu.make_async_copy(w3_any.at[eid], w3b, sc).start()

    def wait_copies(bset):
        w1b, w2b, w3b, sa, sb, sc = bset
        pltpu.make_async_copy(w1_any.at[0], w1b, sa).wait()
        pltpu.make_async_copy(w2_any.at[0], w2b, sb).wait()
        pltpu.make_async_copy(w3_any.at[0], w3b, sc).wait()

    @pl.when(first == 1)
    def _():
        @pl.when(i == 0)
        def _():
            start_copies(meta_ref[0, 0], bufs[0])

        @pl.when(parity == 0)
        def _():
            wait_copies(bufs[0])

        @pl.when(parity == 1)
        def _():
            wait_copies(bufs[1])

        @pl.when((nxt >= 0) & (parity == 0))
        def _():
            start_copies(nxt, bufs[1])

        @pl.when((nxt >= 0) & (parity == 1))
        def _():
            start_copies(nxt, bufs[0])

    def compute(bset):
        w1b, w2b, w3b = bset[0], bset[1], bset[2]
        x = xd_ref[...]
        a = jnp.dot(x, w1b[...], preferred_element_type=jnp.float32)
        b = jnp.dot(x, w2b[...], preferred_element_type=jnp.float32)
        hh = (a * jax.nn.sigmoid(a)) * b
        y = jnp.dot(hh, w3b[...], preferred_element_type=jnp.float32)
        y_ref[...] = y * ws_ref[:, 0:1]

    active = meta_ref[i, 4]

    @pl.when((parity == 0) & (active == 1))
    def _():
        compute(bufs[0])

    @pl.when((parity == 1) & (active == 1))
    def _():
        compute(bufs[1])


def _ffn(xd, w1, w2, w3, wslot, meta, nt):
    total = xd.shape[0]
    return pl.pallas_call(
        _ffn_body,
        grid_spec=pltpu.PrefetchScalarGridSpec(
            num_scalar_prefetch=1,
            grid=(nt,),
            in_specs=[
                pl.BlockSpec((TM, D), lambda i, m: (i, 0)),
                pl.BlockSpec((TM, 128), lambda i, m: (i, 0)),
                pl.BlockSpec(memory_space=pl.ANY),
                pl.BlockSpec(memory_space=pl.ANY),
                pl.BlockSpec(memory_space=pl.ANY),
            ],
            out_specs=pl.BlockSpec((TM, D), lambda i, m: (i, 0)),
            scratch_shapes=[
                pltpu.VMEM((D, DFF), jnp.float32),
                pltpu.VMEM((D, DFF), jnp.float32),
                pltpu.VMEM((DFF, D), jnp.float32),
                pltpu.VMEM((D, DFF), jnp.float32),
                pltpu.VMEM((D, DFF), jnp.float32),
                pltpu.VMEM((DFF, D), jnp.float32),
                pltpu.SemaphoreType.DMA,
                pltpu.SemaphoreType.DMA,
                pltpu.SemaphoreType.DMA,
                pltpu.SemaphoreType.DMA,
                pltpu.SemaphoreType.DMA,
                pltpu.SemaphoreType.DMA,
            ],
        ),
        out_shape=jax.ShapeDtypeStruct((total, D), jnp.float32),
    )(meta, xd, wslot, w1, w2, w3)


def _dispatch(x_flat, slot1, slot2, wt1x, wt2x, total):
    T = x_flat.shape[0]
    per_w = T // NW
    CH = 64
    mesh = plsc.VectorSubcoreMesh(core_axis_name="c", subcore_axis_name="s")

    nch = per_w // CH

    @functools.partial(
        pl.kernel, mesh=mesh,
        out_type=(
            jax.ShapeDtypeStruct((total, D), jnp.float32),
            jax.ShapeDtypeStruct((total, 128), jnp.float32),
        ),
        scratch_types=[
            [pltpu.VMEM((CH,), jnp.int32) for _ in range(nch)],
            [pltpu.VMEM((CH,), jnp.int32) for _ in range(nch)],
            [pltpu.VMEM((CH, D), jnp.float32) for _ in range(nch)],
            [pltpu.VMEM((CH, 128), jnp.float32) for _ in range(nch)],
            [pltpu.VMEM((CH, 128), jnp.float32) for _ in range(nch)],
            [pltpu.SemaphoreType.DMA for _ in range(nch)],
            [pltpu.SemaphoreType.DMA for _ in range(nch)],
            [pltpu.SemaphoreType.DMA for _ in range(nch)],
            [pltpu.SemaphoreType.DMA for _ in range(nch)],
        ],
    )
    def disp(x_hbm, s1_hbm, s2_hbm, wt1_hbm, wt2_hbm, xd_hbm, ws_hbm,
             s1_vs, s2_vs, rows_vs, w1_vs, w2_vs, sems1, sems2, semw1, semw2):
        wid = lax.axis_index("s") * 2 + lax.axis_index("c")
        pend = []
        for cidx in range(nch):
            base = wid * per_w + cidx * CH
            pltpu.sync_copy(s1_hbm.at[pl.ds(base, CH)], s1_vs[cidx])
            pltpu.sync_copy(s2_hbm.at[pl.ds(base, CH)], s2_vs[cidx])
            pltpu.sync_copy(x_hbm.at[pl.ds(base, CH)], rows_vs[cidx])
            pltpu.sync_copy(wt1_hbm.at[pl.ds(base, CH)], w1_vs[cidx])
            pltpu.sync_copy(wt2_hbm.at[pl.ds(base, CH)], w2_vs[cidx])
            pend.append(pltpu.async_copy(rows_vs[cidx], xd_hbm.at[s1_vs[cidx]], sems1[cidx]))
            pend.append(pltpu.async_copy(rows_vs[cidx], xd_hbm.at[s2_vs[cidx]], sems2[cidx]))
            pend.append(pltpu.async_copy(w1_vs[cidx], ws_hbm.at[s1_vs[cidx]], semw1[cidx]))
            pend.append(pltpu.async_copy(w2_vs[cidx], ws_hbm.at[s2_vs[cidx]], semw2[cidx]))
        for cp in pend:
            cp.wait()

    return disp(x_flat, slot1, slot2, wt1x, wt2x)


def _combine(y, slot1, slot2, T):
    per_w = T // NW
    CH = 16
    mesh = plsc.VectorSubcoreMesh(core_axis_name="c", subcore_axis_name="s")

    nch = per_w // CH

    @functools.partial(
        pl.kernel, mesh=mesh,
        out_type=jax.ShapeDtypeStruct((T, D), jnp.float32),
        scratch_types=[
            pltpu.VMEM((per_w,), jnp.int32),
            pltpu.VMEM((per_w,), jnp.int32),
            pltpu.VMEM((CH, D), jnp.float32),
            pltpu.VMEM((CH, D), jnp.float32),
            pltpu.VMEM((CH, D), jnp.float32),
            pltpu.VMEM((CH, D), jnp.float32),
            pltpu.VMEM((CH, D), jnp.float32),
            pltpu.VMEM((CH, D), jnp.float32),
            pltpu.SemaphoreType.DMA,
            pltpu.SemaphoreType.DMA,
            pltpu.SemaphoreType.DMA,
            pltpu.SemaphoreType.DMA,
            pltpu.SemaphoreType.DMA,
            pltpu.SemaphoreType.DMA,
        ],
    )
    def comb(y_hbm, s1_hbm, s2_hbm, out_hbm,
             s1_v, s2_v, b1a, b2a, b1b, b2b, oba, obb,
             g1a, g2a, g1b, g2b, sta, stb):
        wid = lax.axis_index("s") * 2 + lax.axis_index("c")
        base0 = wid * per_w
        pltpu.sync_copy(s1_hbm.at[pl.ds(base0, per_w)], s1_v)
        pltpu.sync_copy(s2_hbm.at[pl.ds(base0, per_w)], s2_v)
        bufs = [(b1a, b2a, oba, g1a, g2a, sta), (b1b, b2b, obb, g1b, g2b, stb)]

        def start(c):
            b1, b2, _, g1, g2, _ = bufs[c % 2]
            cp1 = pltpu.async_copy(y_hbm.at[s1_v.at[pl.ds(c * CH, CH)]], b1, g1)
            cp2 = pltpu.async_copy(y_hbm.at[s2_v.at[pl.ds(c * CH, CH)]], b2, g2)
            return cp1, cp2

        pend = start(0)
        stores = [None, None]
        for c in range(nch):
            nxt = start(c + 1) if c + 1 < nch else None
            pend[0].wait()
            pend[1].wait()
            b1, b2, ob, _, _, stsem = bufs[c % 2]
            if stores[c % 2] is not None:
                stores[c % 2].wait()
            for i in range(CH):

                def body(jj, _, i=i, b1=b1, b2=b2, ob=ob):
                    col = jj * 64
                    for u in range(4):
                        sl = pl.ds(col + u * 16, 16)
                        ob[i, sl] = b1[i, sl] + b2[i, sl]
                    return 0

                lax.fori_loop(0, D // 64, body, 0)
            st = pltpu.async_copy(ob, out_hbm.at[pl.ds(base0 + c * CH, CH)], stsem)
            stores[c % 2] = st
            pend = nxt
        for st in stores:
            if st is not None:
                st.wait()

    return comb(y, slot1, slot2)


def kernel(x, gate_w1, gate_b1, gate_w2, w1, w2, w3):
    b, s, d = x.shape
    T = b * s
    NT = (T * K) // TM + NE
    total = NT * TM
    x_flat = x.reshape(T, d)

    slot1, slot2, wt1x, wt2x, te, bal = _gating(x_flat, gate_w1, gate_b1, gate_w2)
    slot1 = slot1.reshape(T)
    slot2 = slot2.reshape(T)
    tile_eid = te[:, 0]
    tile_act = te[:, 1]

    first = jnp.concatenate([jnp.ones((1,), jnp.int32),
                             (tile_eid[1:] != tile_eid[:-1]).astype(jnp.int32)])
    seg = jnp.cumsum(first) - 1
    parity = seg % 2
    ge = jnp.zeros((NT + 1,), jnp.int32).at[seg].max(tile_eid)
    nxt = jnp.where(seg + 1 <= seg[NT - 1], ge[seg + 1], -1)
    meta = jnp.stack([tile_eid, first, parity, nxt, tile_act], axis=1)

    xd, wslot = _dispatch(x_flat, slot1, slot2, wt1x, wt2x, total)
    y = _ffn(xd, w1, w2, w3, wslot, meta, NT)
    out = _combine(y, slot1, slot2, T)
    return out.reshape(b, s, d), bal[0, 0]
